# Initial kernel scaffold; baseline (speedup 1.0000x reference)
import jax
import jax.numpy as jnp
from jax.experimental import pallas as pl


def kernel(x, edge_index):
    raise NotImplementedError("write your pallas kernel here")


# speedup vs baseline: 6.7679x; 6.7679x over previous
import functools

import jax
import jax.numpy as jnp
from jax import lax
from jax.experimental import pallas as pl
from jax.experimental.pallas import tpu as pltpu
from jax.experimental.pallas import tpu_sc as plsc

N_NODES = 10000
N_EDGES = 320000
D_FEAT = 128

NC = 2
NS = 16
NW = NC * NS

E_PER_W = N_EDGES // NW
CHUNK = 80
N_CHUNKS = E_PER_W // CHUNK
ROWS_PER_S = 624
ROWS_REM = N_NODES - NS * ROWS_PER_S
REM_BASE = NS * ROWS_PER_S

_MESH = plsc.VectorSubcoreMesh(core_axis_name="c", subcore_axis_name="s")


@functools.partial(
    pl.kernel,
    out_type=jax.ShapeDtypeStruct((NC, 2, N_NODES), jnp.float32),
    mesh=_MESH,
    scratch_types=[
        pltpu.VMEM((N_CHUNKS, CHUNK), jnp.int32),
        pltpu.VMEM((N_CHUNKS, CHUNK), jnp.int32),
        pltpu.VMEM((CHUNK,), jnp.float32),
        pltpu.VMEM((2, N_NODES), jnp.float32),
        pltpu.VMEM_SHARED((N_NODES,), jnp.float32),
        pltpu.VMEM_SHARED((N_NODES,), jnp.float32),
    ],
)
def _sc_degrees(src_hbm, dst_hbm, ones_hbm, zeros_hbm, out_hbm,
                sidx, didx, ones_v, stage, deg_s, deg_d):
    c = lax.axis_index("c")
    s = lax.axis_index("s")
    wid = s * NC + c

    @pl.when(s == 0)
    def _():
        pltpu.sync_copy(zeros_hbm, deg_s)
        pltpu.sync_copy(zeros_hbm, deg_d)

    pltpu.sync_copy(src_hbm.at[wid], sidx)
    pltpu.sync_copy(dst_hbm.at[wid], didx)
    pltpu.sync_copy(ones_hbm, ones_v)
    plsc.subcore_barrier()

    def body(j, carry):
        pltpu.sync_copy(ones_v, deg_s.at[sidx.at[j]], add=True)
        pltpu.sync_copy(ones_v, deg_d.at[didx.at[j]], add=True)
        return carry

    lax.fori_loop(0, N_CHUNKS, body, 0)
    plsc.subcore_barrier()

    @pl.when(s == 0)
    def _():
        pltpu.sync_copy(deg_s, stage.at[0])
        pltpu.sync_copy(deg_d, stage.at[1])
        pltpu.sync_copy(stage, out_hbm.at[c])


@functools.partial(
    pl.kernel,
    out_type=jax.ShapeDtypeStruct((NC, N_NODES, D_FEAT), jnp.float32),
    mesh=_MESH,
    scratch_types=[
        pltpu.VMEM((N_CHUNKS, CHUNK), jnp.int32),
        pltpu.VMEM((N_CHUNKS, CHUNK), jnp.int32),
        pltpu.VMEM((CHUNK, D_FEAT), jnp.float32),
        pltpu.VMEM_SHARED((N_NODES, D_FEAT), jnp.float32),
        pltpu.SemaphoreType.DMA,
    ],
)
def _sc_scatter(feat_hbm, src_hbm, dst_hbm, zrows_hbm, out_hbm,
                sidx, didx, rows, acc, sem):
    c = lax.axis_index("c")
    s = lax.axis_index("s")
    wid = s * NC + c

    pltpu.sync_copy(zrows_hbm, acc.at[pl.ds(s * ROWS_PER_S, ROWS_PER_S)])

    @pl.when(s == 0)
    def _():
        pltpu.sync_copy(zrows_hbm.at[pl.ds(0, ROWS_REM)],
                        acc.at[pl.ds(REM_BASE, ROWS_REM)])

    pltpu.sync_copy(src_hbm.at[wid], sidx)
    pltpu.sync_copy(dst_hbm.at[wid], didx)
    plsc.subcore_barrier()

    def body(j, carry):
        pltpu.async_copy(feat_hbm.at[sidx.at[j]], rows, sem).wait()
        pltpu.sync_copy(rows, acc.at[didx.at[j]], add=True)
        return carry

    lax.fori_loop(0, N_CHUNKS, body, 0)
    plsc.subcore_barrier()

    pltpu.sync_copy(acc.at[pl.ds(s * ROWS_PER_S, ROWS_PER_S)],
                    out_hbm.at[c, pl.ds(s * ROWS_PER_S, ROWS_PER_S)])

    @pl.when(s == 0)
    def _():
        pltpu.sync_copy(acc.at[pl.ds(REM_BASE, ROWS_REM)],
                        out_hbm.at[c, pl.ds(REM_BASE, ROWS_REM)])


def _tc_norm_body(x_ref, degs_ref, feat_ref):
    deg = degs_ref[0, 0, :] + degs_ref[1, 0, :]
    norm = lax.rsqrt(jnp.maximum(deg, 1.0))
    feat_ref[...] = x_ref[...] * norm[:, None]


def _tc_final_body(p_ref, degs_ref, out_ref):
    deg = degs_ref[0, 1, :] + degs_ref[1, 1, :]
    norm = lax.rsqrt(jnp.maximum(deg, 1.0))
    out_ref[...] = (p_ref[0] + p_ref[1]) * norm[:, None]


def kernel(x, edge_index):
    src3 = edge_index[0].reshape(NW, N_CHUNKS, CHUNK)
    dst3 = edge_index[1].reshape(NW, N_CHUNKS, CHUNK)
    ones = jnp.ones((CHUNK,), jnp.float32)
    zeros1 = jnp.zeros((N_NODES,), jnp.float32)
    zrows = jnp.zeros((ROWS_PER_S, D_FEAT), jnp.float32)

    degs = _sc_degrees(src3, dst3, ones, zeros1)

    feat = pl.pallas_call(
        _tc_norm_body,
        out_shape=jax.ShapeDtypeStruct((N_NODES, D_FEAT), jnp.float32),
    )(x, degs)

    parts = _sc_scatter(feat, src3, dst3, zrows)

    out = pl.pallas_call(
        _tc_final_body,
        out_shape=jax.ShapeDtypeStruct((N_NODES, D_FEAT), jnp.float32),
    )(parts, degs)
    return out


# speedup vs baseline: 10.0337x; 1.4825x over previous
import functools

import jax
import jax.numpy as jnp
from jax import lax
from jax.experimental import pallas as pl
from jax.experimental.pallas import tpu as pltpu
from jax.experimental.pallas import tpu_sc as plsc

N_NODES = 10000
N_EDGES = 320000
D_FEAT = 128

NC = 2
NS = 16
NW = NC * NS

CHUNK = 128
N_CHUNKS_TOT = N_EDGES // CHUNK
CPW = N_CHUNKS_TOT // NW
NPAIR = CPW // 2
EXTRA_BASE = NW * CPW
N_EXTRA = N_CHUNKS_TOT - EXTRA_BASE

ROWS_PER_S = 624
ROWS_REM = N_NODES - NS * ROWS_PER_S
REM_BASE = NS * ROWS_PER_S

_MESH = plsc.VectorSubcoreMesh(core_axis_name="c", subcore_axis_name="s")


@functools.partial(
    pl.kernel,
    out_type=jax.ShapeDtypeStruct((NC, 2, N_NODES), jnp.float32),
    mesh=_MESH,
    scratch_types=[
        pltpu.VMEM((CPW, 1, CHUNK), jnp.int32),
        pltpu.VMEM((CPW, 1, CHUNK), jnp.int32),
        pltpu.VMEM((CHUNK,), jnp.float32),
        pltpu.VMEM((2, N_NODES), jnp.float32),
        pltpu.VMEM_SHARED((N_NODES,), jnp.float32),
        pltpu.VMEM_SHARED((N_NODES,), jnp.float32),
    ],
)
def _sc_degrees(src_hbm, dst_hbm, ones_hbm, zeros_hbm, out_hbm,
                sidx, didx, ones_v, stage, deg_s, deg_d):
    c = lax.axis_index("c")
    s = lax.axis_index("s")
    wid = s * NC + c

    @pl.when(s == 0)
    def _():
        pltpu.sync_copy(zeros_hbm, deg_s)
        pltpu.sync_copy(zeros_hbm, deg_d)

    pltpu.sync_copy(src_hbm.at[pl.ds(wid * CPW, CPW)], sidx)
    pltpu.sync_copy(dst_hbm.at[pl.ds(wid * CPW, CPW)], didx)
    pltpu.sync_copy(ones_hbm, ones_v)
    plsc.subcore_barrier()

    def body(j, carry):
        pltpu.sync_copy(ones_v, deg_s.at[sidx.at[j, 0]], add=True)
        pltpu.sync_copy(ones_v, deg_d.at[didx.at[j, 0]], add=True)
        return carry

    lax.fori_loop(0, CPW, body, 0)

    @pl.when(wid < N_EXTRA)
    def _():
        pltpu.sync_copy(src_hbm.at[pl.ds(EXTRA_BASE + wid, 1)], sidx.at[pl.ds(0, 1)])
        pltpu.sync_copy(dst_hbm.at[pl.ds(EXTRA_BASE + wid, 1)], didx.at[pl.ds(0, 1)])
        pltpu.sync_copy(ones_v, deg_s.at[sidx.at[0, 0]], add=True)
        pltpu.sync_copy(ones_v, deg_d.at[didx.at[0, 0]], add=True)

    plsc.subcore_barrier()

    @pl.when(s == 0)
    def _():
        pltpu.sync_copy(deg_s, stage.at[0])
        pltpu.sync_copy(deg_d, stage.at[1])
        pltpu.sync_copy(stage, out_hbm.at[c])


@functools.partial(
    pl.kernel,
    out_type=jax.ShapeDtypeStruct((NC, N_NODES, D_FEAT), jnp.float32),
    mesh=_MESH,
    scratch_types=[
        pltpu.VMEM((2, 1, CHUNK), jnp.int32),
        pltpu.VMEM((2, 1, CHUNK), jnp.int32),
        pltpu.VMEM((CHUNK, D_FEAT), jnp.float32),
        pltpu.VMEM((CHUNK, D_FEAT), jnp.float32),
        pltpu.VMEM_SHARED((N_NODES, D_FEAT), jnp.float32),
        pltpu.SemaphoreType.DMA,
        pltpu.SemaphoreType.DMA,
        pltpu.SemaphoreType.DMA,
        pltpu.SemaphoreType.DMA,
    ],
)
def _sc_scatter(feat_hbm, src_hbm, dst_hbm, zrows_hbm, out_hbm,
                sidx, didx, rows_a, rows_b, acc,
                gsem_a, gsem_b, isem_a, isem_b):
    c = lax.axis_index("c")
    s = lax.axis_index("s")
    wid = s * NC + c
    base = wid * CPW

    pltpu.sync_copy(zrows_hbm, acc.at[pl.ds(s * ROWS_PER_S, ROWS_PER_S)])

    @pl.when(s == 0)
    def _():
        pltpu.sync_copy(zrows_hbm.at[pl.ds(0, ROWS_REM)],
                        acc.at[pl.ds(REM_BASE, ROWS_REM)])

    plsc.subcore_barrier()

    def istart(j, b, isem):
        pltpu.async_copy(src_hbm.at[pl.ds(base + j, 1)], sidx.at[pl.ds(b, 1)], isem)
        pltpu.async_copy(dst_hbm.at[pl.ds(base + j, 1)], didx.at[pl.ds(b, 1)], isem)

    def iwait(b, isem):
        pltpu.make_async_copy(src_hbm.at[pl.ds(0, 1)], sidx.at[pl.ds(b, 1)], isem).wait()
        pltpu.make_async_copy(dst_hbm.at[pl.ds(0, 1)], didx.at[pl.ds(b, 1)], isem).wait()

    def gstart(b, buf, gsem):
        pltpu.async_copy(feat_hbm.at[sidx.at[b, 0]], buf, gsem)

    def gwait(buf, gsem):
        pltpu.make_async_copy(feat_hbm.at[sidx.at[0, 0]], buf, gsem).wait()

    def scatter(b, buf):
        pltpu.sync_copy(buf, acc.at[didx.at[b, 0]], add=True)

    istart(0, 0, isem_a)
    iwait(0, isem_a)
    gstart(0, rows_a, gsem_a)
    istart(1, 1, isem_b)

    def body(i, carry):
        j0 = 2 * i
        gwait(rows_a, gsem_a)
        iwait(1, isem_b)
        gstart(1, rows_b, gsem_b)
        scatter(0, rows_a)

        @pl.when(i < NPAIR - 1)
        def _():
            istart(j0 + 2, 0, isem_a)

        gwait(rows_b, gsem_b)

        @pl.when(i < NPAIR - 1)
        def _():
            iwait(0, isem_a)
            gstart(0, rows_a, gsem_a)

        scatter(1, rows_b)

        @pl.when(i < NPAIR - 1)
        def _():
            istart(j0 + 3, 1, isem_b)

        return carry

    lax.fori_loop(0, NPAIR, body, 0)

    @pl.when(wid < N_EXTRA)
    def _():
        pltpu.async_copy(src_hbm.at[pl.ds(EXTRA_BASE + wid, 1)],
                         sidx.at[pl.ds(0, 1)], isem_a)
        pltpu.async_copy(dst_hbm.at[pl.ds(EXTRA_BASE + wid, 1)],
                         didx.at[pl.ds(0, 1)], isem_a)
        iwait(0, isem_a)
        gstart(0, rows_a, gsem_a)
        gwait(rows_a, gsem_a)
        scatter(0, rows_a)

    plsc.subcore_barrier()

    pltpu.sync_copy(acc.at[pl.ds(s * ROWS_PER_S, ROWS_PER_S)],
                    out_hbm.at[c, pl.ds(s * ROWS_PER_S, ROWS_PER_S)])

    @pl.when(s == 0)
    def _():
        pltpu.sync_copy(acc.at[pl.ds(REM_BASE, ROWS_REM)],
                        out_hbm.at[c, pl.ds(REM_BASE, ROWS_REM)])


def _tc_norm_body(x_ref, degs_ref, feat_ref):
    deg = degs_ref[0, 0, :] + degs_ref[1, 0, :]
    norm = lax.rsqrt(jnp.maximum(deg, 1.0))
    feat_ref[...] = x_ref[...] * norm[:, None]


def _tc_final_body(p_ref, degs_ref, out_ref):
    deg = degs_ref[0, 1, :] + degs_ref[1, 1, :]
    norm = lax.rsqrt(jnp.maximum(deg, 1.0))
    out_ref[...] = (p_ref[0] + p_ref[1]) * norm[:, None]


def kernel(x, edge_index):
    src3 = edge_index[0].reshape(N_CHUNKS_TOT, 1, CHUNK)
    dst3 = edge_index[1].reshape(N_CHUNKS_TOT, 1, CHUNK)
    ones = jnp.ones((CHUNK,), jnp.float32)
    zeros1 = jnp.zeros((N_NODES,), jnp.float32)
    zrows = jnp.zeros((ROWS_PER_S, D_FEAT), jnp.float32)

    degs = _sc_degrees(src3, dst3, ones, zeros1)

    feat = pl.pallas_call(
        _tc_norm_body,
        out_shape=jax.ShapeDtypeStruct((N_NODES, D_FEAT), jnp.float32),
    )(x, degs)

    parts = _sc_scatter(feat, src3, dst3, zrows)

    out = pl.pallas_call(
        _tc_final_body,
        out_shape=jax.ShapeDtypeStruct((N_NODES, D_FEAT), jnp.float32),
    )(parts, degs)
    return out
